# baseline (speedup 1.0000x reference)
import functools

import jax
import jax.numpy as jnp
from jax import lax
from jax.experimental import pallas as pl
from jax.experimental.pallas import tpu as pltpu
from jax.experimental.pallas import tpu_sc as plsc

B = 4
NY = 512
NX = 512
C = 64
S = B * NY * NX
P = 40000
L = 16
NC = 2
NS = 16
NW = NC * NS
CHUNK = 128
NUM_CHUNKS = (P + NW * CHUNK - 1) // (NW * CHUNK) * NW
KMAX = NUM_CHUNKS // NW


def _sc_scatter_body(vf_hbm, b_hbm, y_hbm, x_hbm, out_hbm,
                     cbuf, dbuf, ibuf, sem_in, sem_sc):
    cid = lax.axis_index("c")
    sid = lax.axis_index("s")
    w = sid * NC + cid

    in_copies = []
    for k in range(KMAX):
        chunk = w + NW * k
        start = jnp.minimum(chunk * CHUNK, P - CHUNK)
        for j, col in enumerate((b_hbm, y_hbm, x_hbm)):
            in_copies.append(
                pltpu.async_copy(col.at[pl.ds(start, CHUNK)],
                                 cbuf.at[k, j], sem_in))
        in_copies.append(
            pltpu.async_copy(vf_hbm.at[pl.ds(start, CHUNK)], dbuf.at[k], sem_in))
    for cp in in_copies:
        cp.wait()

    sc_copies = []
    for k in range(KMAX):
        for g in range(CHUNK // L):
            bv = cbuf[k, 0, pl.ds(g * L, L)]
            yv = cbuf[k, 1, pl.ds(g * L, L)]
            xv = cbuf[k, 2, pl.ds(g * L, L)]
            lin = jnp.minimum(bv, B - 1) * (NY * NX) + yv * NX + xv
            ibuf[k, pl.ds(g * L, L)] = lin
        sc_copies.append(
            pltpu.async_copy(dbuf.at[k], out_hbm.at[ibuf.at[k]], sem_sc))
    for cp in sc_copies:
        cp.wait()


def _sc_scatter(vf, bcol, ycol, xcol):
    mesh = plsc.VectorSubcoreMesh(core_axis_name="c", subcore_axis_name="s")
    kfn = pl.kernel(
        _sc_scatter_body,
        mesh=mesh,
        out_type=(),
        compiler_params=pltpu.CompilerParams(use_tc_tiling_on_sc=False,
                                             skip_device_barrier=True),
        scratch_types=[
            pltpu.VMEM((KMAX, 3, CHUNK), jnp.int32),
            pltpu.VMEM((KMAX, CHUNK, C), jnp.float32),
            pltpu.VMEM((KMAX, CHUNK), jnp.int32),
            pltpu.SemaphoreType.DMA,
            pltpu.SemaphoreType.DMA,
        ],
    )
    canvas_ref = jax.new_ref(jnp.zeros((S, C), jnp.float32))
    kfn(vf, bcol, ycol, xcol, canvas_ref)
    return canvas_ref[...]


YB = 8


def _tc_transpose_body(x_ref, o_ref):
    for y in range(YB):
        o_ref[:, y, :] = jnp.transpose(x_ref[0, y], (1, 0))


def _tc_transpose(canvas4):
    return pl.pallas_call(
        _tc_transpose_body,
        grid=(B, NY // YB),
        in_specs=[pl.BlockSpec((1, YB, NX, C), lambda b, y: (b, y, 0, 0))],
        out_specs=pl.BlockSpec((C, YB, NX), lambda b, y: (b, y, 0)),
        out_shape=jax.ShapeDtypeStruct((B * C, NY, NX), jnp.float32),
    )(canvas4)


def kernel(voxel_features, coords, batch_size, input_shape):
    del batch_size, input_shape
    canvas = _sc_scatter(voxel_features, coords[:, 0], coords[:, 2], coords[:, 3])
    out = _tc_transpose(canvas.reshape(B, NY, NX, C))
    return out.reshape(B, C, NY, NX)


# speedup vs baseline: 1.5270x; 1.5270x over previous
import functools

import jax
import jax.numpy as jnp
from jax import lax
from jax.experimental import pallas as pl
from jax.experimental.pallas import tpu as pltpu
from jax.experimental.pallas import tpu_sc as plsc

B = 4
NY = 512
NX = 512
C = 64
S = B * NY * NX
P = 40000
L = 16
NC = 2
NS = 16
NW = NC * NS
CW = 128
CHUNK = 128
NUM_CHUNKS = (P + NW * CHUNK - 1) // (NW * CHUNK) * NW
KMAX = NUM_CHUNKS // NW
WAVE = 5


def _sc_scatter_body(vf_hbm, b_hbm, y_hbm, x_hbm, out_hbm,
                     cbuf, dbuf, ibuf, sem_in, sem_sc):
    cid = lax.axis_index("c")
    sid = lax.axis_index("s")
    w = sid * NC + cid

    for wave in range(KMAX // WAVE):
        in_copies = []
        for kk in range(WAVE):
            k = wave * WAVE + kk
            chunk = w + NW * k
            start = jnp.minimum(chunk * CHUNK, P - CHUNK)
            for j, col in enumerate((b_hbm, y_hbm, x_hbm)):
                in_copies.append(
                    pltpu.async_copy(col.at[pl.ds(start, CHUNK)],
                                     cbuf.at[k, j], sem_in))
            in_copies.append(
                pltpu.async_copy(vf_hbm.at[pl.ds(start, CHUNK)],
                                 dbuf.at[kk, :, pl.ds(0, C)], sem_in))
        for cp in in_copies:
            cp.wait()

        sc_copies = []
        for kk in range(WAVE):
            k = wave * WAVE + kk
            for g in range(CHUNK // L):
                bv = cbuf[k, 0, pl.ds(g * L, L)]
                yv = cbuf[k, 1, pl.ds(g * L, L)]
                xv = cbuf[k, 2, pl.ds(g * L, L)]
                lin = jnp.minimum(bv, B - 1) * (NY * NX) + yv * NX + xv
                ibuf[k, pl.ds(g * L, L)] = lin
            sc_copies.append(
                pltpu.async_copy(dbuf.at[kk], out_hbm.at[ibuf.at[k]], sem_sc))
        for cp in sc_copies:
            cp.wait()


def _sc_scatter(vf, bcol, ycol, xcol):
    mesh = plsc.VectorSubcoreMesh(core_axis_name="c", subcore_axis_name="s")
    kfn = pl.kernel(
        _sc_scatter_body,
        mesh=mesh,
        out_type=(),
        compiler_params=pltpu.CompilerParams(use_tc_tiling_on_sc=False,
                                             skip_device_barrier=True),
        scratch_types=[
            pltpu.VMEM((KMAX, 3, CHUNK), jnp.int32),
            pltpu.VMEM((WAVE, CHUNK, CW), jnp.float32),
            pltpu.VMEM((KMAX, CHUNK), jnp.int32),
            pltpu.SemaphoreType.DMA,
            pltpu.SemaphoreType.DMA,
        ],
    )
    canvas_ref = jax.new_ref(jnp.zeros((S, CW), jnp.float32))
    kfn(vf, bcol, ycol, xcol, canvas_ref)
    return canvas_ref[...]


YB = 8


def _tc_transpose_body(x_ref, o_ref):
    for y in range(YB):
        o_ref[:, y, :] = jnp.transpose(x_ref[0, y, :, :C], (1, 0))


def _tc_transpose(canvas4):
    return pl.pallas_call(
        _tc_transpose_body,
        grid=(B, NY // YB),
        in_specs=[pl.BlockSpec((1, YB, NX, CW), lambda b, y: (b, y, 0, 0))],
        out_specs=pl.BlockSpec((C, YB, NX), lambda b, y: (b, y, 0)),
        out_shape=jax.ShapeDtypeStruct((B * C, NY, NX), jnp.float32),
    )(canvas4)


def kernel(voxel_features, coords, batch_size, input_shape):
    del batch_size, input_shape
    canvas = _sc_scatter(voxel_features, coords[:, 0], coords[:, 2], coords[:, 3])
    out = _tc_transpose(canvas.reshape(B, NY, NX, CW))
    return out.reshape(B, C, NY, NX)


# speedup vs baseline: 1.7619x; 1.1538x over previous
import functools

import jax
import jax.numpy as jnp
from jax import lax
from jax.experimental import pallas as pl
from jax.experimental.pallas import tpu as pltpu
from jax.experimental.pallas import tpu_sc as plsc

B = 4
NY = 512
NX = 512
C = 64
S = B * NY * NX
P = 40000
L = 16
NC = 2
NS = 16
NW = NC * NS
CW = 128
CHUNK = 128
NUM_CHUNKS = (P + NW * CHUNK - 1) // (NW * CHUNK) * NW
KMAX = NUM_CHUNKS // NW
WAVE = 5


def _sc_scatter_body(vf_hbm, b_hbm, y_hbm, x_hbm, out_hbm,
                     cbuf, dbuf, ibuf, sem_in, sem_sc):
    cid = lax.axis_index("c")
    sid = lax.axis_index("s")
    w = sid * NC + cid

    for wave in range(KMAX // WAVE):
        in_copies = []
        for kk in range(WAVE):
            k = wave * WAVE + kk
            chunk = w + NW * k
            start = jnp.minimum(chunk * CHUNK, P - CHUNK)
            for j, col in enumerate((b_hbm, y_hbm, x_hbm)):
                in_copies.append(
                    pltpu.async_copy(col.at[pl.ds(start, CHUNK)],
                                     cbuf.at[k, j], sem_in))
            in_copies.append(
                pltpu.async_copy(vf_hbm.at[pl.ds(start, CHUNK)],
                                 dbuf.at[kk, :, pl.ds(0, C)], sem_in))
        for cp in in_copies:
            cp.wait()

        sc_copies = []
        for kk in range(WAVE):
            k = wave * WAVE + kk
            for g in range(CHUNK // L):
                bv = cbuf[k, 0, pl.ds(g * L, L)]
                yv = cbuf[k, 1, pl.ds(g * L, L)]
                xv = cbuf[k, 2, pl.ds(g * L, L)]
                lin = jnp.minimum(bv, B - 1) * (NY * NX) + yv * NX + xv
                ibuf[k, pl.ds(g * L, L)] = lin
            sc_copies.append(
                pltpu.async_copy(dbuf.at[kk], out_hbm.at[ibuf.at[k]], sem_sc))
        for cp in sc_copies:
            cp.wait()


def _sc_scatter(vf, bcol, ycol, xcol):
    mesh = plsc.VectorSubcoreMesh(core_axis_name="c", subcore_axis_name="s")
    kfn = pl.kernel(
        _sc_scatter_body,
        mesh=mesh,
        out_type=(),
        compiler_params=pltpu.CompilerParams(use_tc_tiling_on_sc=False,
                                             skip_device_barrier=True),
        scratch_types=[
            pltpu.VMEM((KMAX, 3, CHUNK), jnp.int32),
            pltpu.VMEM((WAVE, CHUNK, CW), jnp.float32),
            pltpu.VMEM((KMAX, CHUNK), jnp.int32),
            pltpu.SemaphoreType.DMA,
            pltpu.SemaphoreType.DMA,
        ],
    )
    canvas_ref = jax.new_ref(jnp.zeros((S, CW), jnp.float32))
    kfn(vf, bcol, ycol, xcol, canvas_ref)
    return canvas_ref[...]


YB = 16


def _tc_transpose_body(x_ref, o_ref):
    for y in range(YB):
        o_ref[:, y, :] = jnp.transpose(x_ref[0, y, :, :C], (1, 0))


def _tc_transpose(canvas4):
    return pl.pallas_call(
        _tc_transpose_body,
        grid=(B, NY // YB),
        in_specs=[pl.BlockSpec((1, YB, NX, CW), lambda b, y: (b, y, 0, 0))],
        out_specs=pl.BlockSpec((C, YB, NX), lambda b, y: (b, y, 0)),
        out_shape=jax.ShapeDtypeStruct((B * C, NY, NX), jnp.float32),
    )(canvas4)


def kernel(voxel_features, coords, batch_size, input_shape):
    del batch_size, input_shape
    canvas = _sc_scatter(voxel_features, coords[:, 0], coords[:, 2], coords[:, 3])
    out = _tc_transpose(canvas.reshape(B, NY, NX, CW))
    return out.reshape(B, C, NY, NX)


# speedup vs baseline: 1.8366x; 1.0424x over previous
import functools

import jax
import jax.numpy as jnp
from jax import lax
from jax.experimental import pallas as pl
from jax.experimental.pallas import tpu as pltpu
from jax.experimental.pallas import tpu_sc as plsc

B = 4
NY = 512
NX = 512
C = 64
S = B * NY * NX
P = 40000
L = 16
NC = 2
NS = 16
NW = NC * NS
CW = 128
CHUNK = 128
NUM_CHUNKS = (P + NW * CHUNK - 1) // (NW * CHUNK) * NW
KMAX = NUM_CHUNKS // NW
WAVE = 5


def _sc_scatter_body(vf_hbm, b_hbm, y_hbm, x_hbm, out_hbm,
                     cbuf, dbuf, ibuf, sem_in, sem_sc):
    cid = lax.axis_index("c")
    sid = lax.axis_index("s")
    w = sid * NC + cid

    for wave in range(KMAX // WAVE):
        in_copies = []
        for kk in range(WAVE):
            k = wave * WAVE + kk
            chunk = w + NW * k
            start = jnp.minimum(chunk * CHUNK, P - CHUNK)
            for j, col in enumerate((b_hbm, y_hbm, x_hbm)):
                in_copies.append(
                    pltpu.async_copy(col.at[pl.ds(start, CHUNK)],
                                     cbuf.at[k, j], sem_in))
            in_copies.append(
                pltpu.async_copy(vf_hbm.at[pl.ds(start, CHUNK)],
                                 dbuf.at[kk, :, pl.ds(0, C)], sem_in))
        for cp in in_copies:
            cp.wait()

        sc_copies = []
        for kk in range(WAVE):
            k = wave * WAVE + kk
            for g in range(CHUNK // L):
                bv = cbuf[k, 0, pl.ds(g * L, L)]
                yv = cbuf[k, 1, pl.ds(g * L, L)]
                xv = cbuf[k, 2, pl.ds(g * L, L)]
                lin = jnp.minimum(bv, B - 1) * (NY * NX) + yv * NX + xv
                ibuf[k, pl.ds(g * L, L)] = lin
            sc_copies.append(
                pltpu.async_copy(dbuf.at[kk], out_hbm.at[ibuf.at[k]], sem_sc))
        for cp in sc_copies:
            cp.wait()


def _sc_scatter(vf, bcol, ycol, xcol):
    mesh = plsc.VectorSubcoreMesh(core_axis_name="c", subcore_axis_name="s")
    kfn = pl.kernel(
        _sc_scatter_body,
        mesh=mesh,
        out_type=(),
        compiler_params=pltpu.CompilerParams(use_tc_tiling_on_sc=False,
                                             skip_device_barrier=True),
        scratch_types=[
            pltpu.VMEM((KMAX, 3, CHUNK), jnp.int32),
            pltpu.VMEM((WAVE, CHUNK, CW), jnp.float32),
            pltpu.VMEM((KMAX, CHUNK), jnp.int32),
            pltpu.SemaphoreType.DMA,
            pltpu.SemaphoreType.DMA,
        ],
    )
    canvas_ref = jax.new_ref(jnp.zeros((S, CW), jnp.float32))
    kfn(vf, bcol, ycol, xcol, canvas_ref)
    return canvas_ref[...]


YB = 32


def _tc_transpose_body(x_ref, o_ref):
    for y in range(YB):
        o_ref[:, y, :] = jnp.transpose(x_ref[0, y, :, :C], (1, 0))


def _tc_transpose(canvas4):
    return pl.pallas_call(
        _tc_transpose_body,
        grid=(B, NY // YB),
        in_specs=[pl.BlockSpec((1, YB, NX, CW), lambda b, y: (b, y, 0, 0))],
        out_specs=pl.BlockSpec((C, YB, NX), lambda b, y: (b, y, 0)),
        out_shape=jax.ShapeDtypeStruct((B * C, NY, NX), jnp.float32),
    )(canvas4)


def kernel(voxel_features, coords, batch_size, input_shape):
    del batch_size, input_shape
    canvas = _sc_scatter(voxel_features, coords[:, 0], coords[:, 2], coords[:, 3])
    out = _tc_transpose(canvas.reshape(B, NY, NX, CW))
    return out.reshape(B, C, NY, NX)


# speedup vs baseline: 1.8637x; 1.0147x over previous
import functools

import jax
import jax.numpy as jnp
from jax import lax
from jax.experimental import pallas as pl
from jax.experimental.pallas import tpu as pltpu
from jax.experimental.pallas import tpu_sc as plsc

B = 4
NY = 512
NX = 512
C = 64
S = B * NY * NX
P = 40000
L = 16
NC = 2
NS = 16
NW = NC * NS
CW = 128
CHUNK = 128
NUM_CHUNKS = (P + NW * CHUNK - 1) // (NW * CHUNK) * NW
KMAX = NUM_CHUNKS // NW
WAVE = 5


def _sc_scatter_body(vf_hbm, b_hbm, y_hbm, x_hbm, out_hbm,
                     cbuf, dbuf, ibuf, sem_in, sem_sc):
    cid = lax.axis_index("c")
    sid = lax.axis_index("s")
    w = sid * NC + cid

    for wave in range(KMAX // WAVE):
        in_copies = []
        for kk in range(WAVE):
            k = wave * WAVE + kk
            chunk = w + NW * k
            start = jnp.minimum(chunk * CHUNK, P - CHUNK)
            for j, col in enumerate((b_hbm, y_hbm, x_hbm)):
                in_copies.append(
                    pltpu.async_copy(col.at[pl.ds(start, CHUNK)],
                                     cbuf.at[k, j], sem_in))
            in_copies.append(
                pltpu.async_copy(vf_hbm.at[pl.ds(start, CHUNK)],
                                 dbuf.at[kk, :, pl.ds(0, C)], sem_in))
        for cp in in_copies:
            cp.wait()

        sc_copies = []
        for kk in range(WAVE):
            k = wave * WAVE + kk
            for g in range(CHUNK // L):
                bv = cbuf[k, 0, pl.ds(g * L, L)]
                yv = cbuf[k, 1, pl.ds(g * L, L)]
                xv = cbuf[k, 2, pl.ds(g * L, L)]
                lin = jnp.minimum(bv, B - 1) * (NY * NX) + yv * NX + xv
                ibuf[k, pl.ds(g * L, L)] = lin
            sc_copies.append(
                pltpu.async_copy(dbuf.at[kk], out_hbm.at[ibuf.at[k]], sem_sc))
        for cp in sc_copies:
            cp.wait()


def _sc_scatter(vf, bcol, ycol, xcol):
    mesh = plsc.VectorSubcoreMesh(core_axis_name="c", subcore_axis_name="s")
    kfn = pl.kernel(
        _sc_scatter_body,
        mesh=mesh,
        out_type=(),
        compiler_params=pltpu.CompilerParams(use_tc_tiling_on_sc=False,
                                             skip_device_barrier=True),
        scratch_types=[
            pltpu.VMEM((KMAX, 3, CHUNK), jnp.int32),
            pltpu.VMEM((WAVE, CHUNK, CW), jnp.float32),
            pltpu.VMEM((KMAX, CHUNK), jnp.int32),
            pltpu.SemaphoreType.DMA,
            pltpu.SemaphoreType.DMA,
        ],
    )
    canvas_ref = jax.new_ref(jnp.zeros((S, CW), jnp.float32))
    kfn(vf, bcol, ycol, xcol, canvas_ref)
    return canvas_ref[...]


YB = 64


def _tc_transpose_body(x_ref, o_ref):
    for y in range(YB):
        o_ref[:, y, :] = jnp.transpose(x_ref[0, y, :, :C], (1, 0))


def _tc_transpose(canvas4):
    return pl.pallas_call(
        _tc_transpose_body,
        grid=(B, NY // YB),
        in_specs=[pl.BlockSpec((1, YB, NX, CW), lambda b, y: (b, y, 0, 0))],
        out_specs=pl.BlockSpec((C, YB, NX), lambda b, y: (b, y, 0)),
        out_shape=jax.ShapeDtypeStruct((B * C, NY, NX), jnp.float32),
    )(canvas4)


def kernel(voxel_features, coords, batch_size, input_shape):
    del batch_size, input_shape
    canvas = _sc_scatter(voxel_features, coords[:, 0], coords[:, 2], coords[:, 3])
    out = _tc_transpose(canvas.reshape(B, NY, NX, CW))
    return out.reshape(B, C, NY, NX)


# speedup vs baseline: 1.8659x; 1.0012x over previous
import functools

import jax
import jax.numpy as jnp
from jax import lax
from jax.experimental import pallas as pl
from jax.experimental.pallas import tpu as pltpu
from jax.experimental.pallas import tpu_sc as plsc

B = 4
NY = 512
NX = 512
C = 64
S = B * NY * NX
P = 40000
L = 16
NC = 2
NS = 16
NW = NC * NS
CW = 128
CHUNK = 128
NUM_CHUNKS = (P + NW * CHUNK - 1) // (NW * CHUNK) * NW
KMAX = NUM_CHUNKS // NW
WAVE = 5


def _sc_scatter_body(vf_hbm, b_hbm, y_hbm, x_hbm, out_hbm,
                     cbuf, dbuf, ibuf, sem_in, sem_sc):
    cid = lax.axis_index("c")
    sid = lax.axis_index("s")
    w = sid * NC + cid

    for wave in range(KMAX // WAVE):
        in_copies = []
        for kk in range(WAVE):
            k = wave * WAVE + kk
            chunk = w + NW * k
            start = jnp.minimum(chunk * CHUNK, P - CHUNK)
            for j, col in enumerate((b_hbm, y_hbm, x_hbm)):
                in_copies.append(
                    pltpu.async_copy(col.at[pl.ds(start, CHUNK)],
                                     cbuf.at[k, j], sem_in))
            in_copies.append(
                pltpu.async_copy(vf_hbm.at[pl.ds(start, CHUNK)],
                                 dbuf.at[kk, :, pl.ds(0, C)], sem_in))
        for cp in in_copies:
            cp.wait()

        sc_copies = []
        for kk in range(WAVE):
            k = wave * WAVE + kk
            for g in range(CHUNK // L):
                bv = cbuf[k, 0, pl.ds(g * L, L)]
                yv = cbuf[k, 1, pl.ds(g * L, L)]
                xv = cbuf[k, 2, pl.ds(g * L, L)]
                lin = jnp.minimum(bv, B - 1) * (NY * NX) + yv * NX + xv
                ibuf[k, pl.ds(g * L, L)] = lin
            sc_copies.append(
                pltpu.async_copy(dbuf.at[kk], out_hbm.at[ibuf.at[k]], sem_sc))
        for cp in sc_copies:
            cp.wait()


def _sc_scatter(vf, bcol, ycol, xcol):
    mesh = plsc.VectorSubcoreMesh(core_axis_name="c", subcore_axis_name="s")
    kfn = pl.kernel(
        _sc_scatter_body,
        mesh=mesh,
        out_type=(),
        compiler_params=pltpu.CompilerParams(use_tc_tiling_on_sc=False),
        scratch_types=[
            pltpu.VMEM((KMAX, 3, CHUNK), jnp.int32),
            pltpu.VMEM((WAVE, CHUNK, CW), jnp.float32),
            pltpu.VMEM((KMAX, CHUNK), jnp.int32),
            pltpu.SemaphoreType.DMA,
            pltpu.SemaphoreType.DMA,
        ],
    )
    canvas_ref = jax.new_ref(jnp.zeros((S, CW), jnp.float32))
    kfn(vf, bcol, ycol, xcol, canvas_ref)
    return canvas_ref[...]


YB = 64


def _tc_transpose_body(x_ref, o_ref):
    for y in range(YB):
        o_ref[:, y, :] = jnp.transpose(x_ref[0, y, :, :C], (1, 0))


def _tc_transpose(canvas4):
    return pl.pallas_call(
        _tc_transpose_body,
        grid=(B, NY // YB),
        in_specs=[pl.BlockSpec((1, YB, NX, CW), lambda b, y: (b, y, 0, 0))],
        out_specs=pl.BlockSpec((C, YB, NX), lambda b, y: (b, y, 0)),
        out_shape=jax.ShapeDtypeStruct((B * C, NY, NX), jnp.float32),
    )(canvas4)


def kernel(voxel_features, coords, batch_size, input_shape):
    del batch_size, input_shape
    canvas = _sc_scatter(voxel_features, coords[:, 0], coords[:, 2], coords[:, 3])
    out = _tc_transpose(canvas.reshape(B, NY, NX, CW))
    return out.reshape(B, C, NY, NX)
